# Initial kernel scaffold; baseline (speedup 1.0000x reference)
import jax
import jax.numpy as jnp
from jax.experimental import pallas as pl


def kernel(x):
    raise NotImplementedError("write your pallas kernel here")


# speedup vs baseline: 8.9110x; 8.9110x over previous
import jax
import jax.numpy as jnp
from jax import lax
from jax.experimental import pallas as pl
from jax.experimental.pallas import tpu as pltpu

_K = 128
_N = 32768
_RPB = 8

def _body(x_ref, o_ref, key_ref, j_ref):
    _MININT = jnp.int32(-2147483648)
    _MAXINT = jnp.int32(2147483647)
    x = x_ref[...]
    s = lax.bitcast_convert_type(x, jnp.int32)
    key = jnp.where(s >= 0, s, _MININT - (s + jnp.int32(1)))
    key_ref[...] = key

    def count_ge(th):
        return jnp.sum((key >= th).astype(jnp.int32), axis=1, keepdims=True)

    ge0 = count_ge(jnp.zeros((_RPB, 1), jnp.int32)) >= _K
    lo = jnp.where(ge0, jnp.int32(0), _MININT)
    hi = jnp.where(ge0, _MAXINT, jnp.int32(-1))

    def step(_, carry):
        lo, hi = carry
        d = hi - lo
        mid = lo + (d >> 1) + (d & 1)
        ge = count_ge(mid) >= _K
        return jnp.where(ge, mid, lo), jnp.where(ge, hi, mid - 1)

    lo, hi = lax.fori_loop(0, 31, step, (lo, hi))
    t = lo

    cnt_ge = count_ge(t)
    j_ref[...] = jnp.full((_RPB, 1), jnp.int32(_N - 1))

    col = lax.broadcasted_iota(jnp.int32, (_RPB, _N), 1)

    @pl.when(jnp.any(cnt_ge > _K))
    def _resolve_ties():
        eq = key == t
        cnt_eq = jnp.sum(eq.astype(jnp.int32), axis=1, keepdims=True)
        m = _K - (cnt_ge - cnt_eq)

        def jstep(_, carry):
            jlo, jhi = carry
            mid = (jlo + jhi) >> 1
            c = jnp.sum((eq & (col <= mid)).astype(jnp.int32),
                        axis=1, keepdims=True)
            p = c >= m
            return jnp.where(p, jlo, mid + 1), jnp.where(p, mid, jhi)

        jlo, _ = lax.fori_loop(
            0, 15, jstep,
            (jnp.zeros((_RPB, 1), jnp.int32),
             jnp.full((_RPB, 1), jnp.int32(_N - 1))))
        j_ref[...] = jlo

    j = j_ref[...]
    keep = (key > t) | ((key == t) & (col <= j))
    o_ref[...] = jnp.where(keep, jnp.maximum(x, 0.0), 0.0)


def kernel(x):
    rows = x.shape[0]
    return pl.pallas_call(
        _body,
        grid=(rows // _RPB,),
        in_specs=[pl.BlockSpec((_RPB, _N), lambda i: (i, 0))],
        out_specs=pl.BlockSpec((_RPB, _N), lambda i: (i, 0)),
        out_shape=jax.ShapeDtypeStruct(x.shape, x.dtype),
        scratch_shapes=[
            pltpu.VMEM((_RPB, _N), jnp.int32),
            pltpu.VMEM((_RPB, 1), jnp.int32),
        ],
    )(x)
